# baseline (speedup 1.0000x reference)
import functools

import jax
import jax.numpy as jnp
from jax import lax
from jax.experimental import pallas as pl
from jax.experimental.pallas import tpu as pltpu
from jax.experimental.pallas import tpu_sc as plsc


@functools.lru_cache(maxsize=None)
def _make_detile(V, D):
    info = plsc.get_sparse_core_info()
    NC, NS, L = info.num_cores, info.num_subcores, info.num_lanes
    NW = NC * NS
    NBF = V // 128
    Vp = NBF * 128 + 128
    mesh = plsc.VectorSubcoreMesh(core_axis_name="c", subcore_axis_name="s")

    @functools.partial(
        pl.kernel,
        mesh=mesh,
        out_type=jax.ShapeDtypeStruct((Vp, 128), jnp.float32),
        scratch_types=[
            pltpu.VMEM((2, D, 128), jnp.float32),
            pltpu.VMEM((2, 128, 128), jnp.float32),
            [pltpu.SemaphoreType.DMA] * 2,
            [pltpu.SemaphoreType.DMA] * 2,
        ],
        compiler_params=pltpu.CompilerParams(use_tc_tiling_on_sc=True, needs_layout_passes=False),
    )
    def detile_kernel(emb_t, tail_pad, out, stage_v, tout_v, gsems, osems):
        wid = lax.axis_index("s") * NC + lax.axis_index("c")

        @pl.when(wid == 0)
        def _():
            pltpu.sync_copy(tail_pad, tout_v.at[0])
            pltpu.sync_copy(tout_v.at[0], out.at[pl.ds(NBF * 128, 128)])

        nb = (NBF + NW - 1) // NW

        def col(g):
            return lax.rem(wid + g * NW, NBF)

        def src_slice(b):
            return emb_t.at[:, pl.ds(b * 128, 128)]

        def start_in(b, slot):
            pltpu.async_copy(src_slice(b), stage_v.at[slot], gsems[slot])

        def wait_in(b, slot):
            pltpu.make_async_copy(
                src_slice(b), stage_v.at[slot], gsems[slot]
            ).wait()

        def dst_slice(b):
            return out.at[pl.ds(b * 128, 128)]

        def start_out(b, slot):
            pltpu.async_copy(tout_v.at[slot], dst_slice(b), osems[slot])

        def wait_out(b, slot):
            pltpu.make_async_copy(
                tout_v.at[slot], dst_slice(b), osems[slot]
            ).wait()

        lane = lax.iota(jnp.int32, L)
        rowv = [lane + (fg * L) for fg in range(D // L)]
        zero = jnp.zeros((L,), jnp.int32)

        def transpose(slot):
            for l in range(128):
                colv = zero + l
                for fg in range(D // L):
                    vals = plsc.load_gather(stage_v.at[slot], [rowv[fg], colv])
                    tout_v[slot, l, pl.ds(fg * L, L)] = vals

        start_in(col(0), 0)
        start_in(col(1), 1)

        assert nb % 2 == 0 or nb % 2 == 1
        n_groups = nb // 2

        def body(grp, carry):
            g0 = grp * 2
            for b in range(2):
                g = g0 + b
                wait_in(col(g), b)

                @pl.when(grp > 0)
                def _():
                    wait_out(col(g - 2), b)

                transpose(b)
                start_out(col(g), b)

                @pl.when(g + 2 < nb)
                def _():
                    start_in(col(g + 2), b)

            return carry

        lax.fori_loop(0, n_groups, body, 0)
        if nb % 2 == 1:
            g = nb - 1
            wait_in(col(g), 0)
            wait_out(col(g - 2), 0)
            transpose(0)
            start_out(col(g), 0)
            wait_out(col(g - 1), 1)
            wait_out(col(g), 0)
        else:
            wait_out(col(nb - 2), 0)
            wait_out(col(nb - 1), 1)

    return detile_kernel


@functools.lru_cache(maxsize=None)
def _make_gather(V, D, R, S, Vp):
    info = plsc.get_sparse_core_info()
    NC, NS = info.num_cores, info.num_subcores
    NW = NC * NS
    assert R % (NW * 128) == 0
    mesh = plsc.VectorSubcoreMesh(core_axis_name="c", subcore_axis_name="s")

    @functools.partial(
        pl.kernel,
        mesh=mesh,
        out_type=jax.ShapeDtypeStruct((S, R, 128), jnp.float32),
        scratch_types=[
            pltpu.VMEM((4, 128), jnp.int32),
            pltpu.VMEM((4, 128, 128), jnp.float32),
            [pltpu.SemaphoreType.DMA] * 4,
            [pltpu.SemaphoreType.DMA] * 4,
            [pltpu.SemaphoreType.DMA] * 4,
        ],
        compiler_params=pltpu.CompilerParams(use_tc_tiling_on_sc=True, needs_layout_passes=False),
    )
    def gather_kernel(idx_hbm, table_hbm, out_hbm, idx_v, rows_v, isems,
                      gsems, osems):
        wid = lax.axis_index("s") * NC + lax.axis_index("c")

        def idx_slice(s):
            return idx_hbm.at[pl.ds(s * R + wid * 128, 128)]

        def start_idx(s, ib):
            pltpu.async_copy(idx_slice(s), idx_v.at[ib], isems[ib])

        def wait_idx(s, ib):
            pltpu.make_async_copy(idx_slice(s), idx_v.at[ib], isems[ib]).wait()

        def start_gather(ib):
            pltpu.async_copy(
                table_hbm.at[idx_v.at[ib]], rows_v.at[ib], gsems[ib]
            )

        def wait_gather(ib):
            pltpu.make_async_copy(
                table_hbm.at[idx_v.at[ib]], rows_v.at[ib], gsems[ib]
            ).wait()

        def out_slice(s):
            return out_hbm.at[s, pl.ds(wid * 128, 128), :]

        def start_out(s, b):
            pltpu.async_copy(rows_v.at[b], out_slice(s), osems[b])

        def wait_out(s, b):
            pltpu.make_async_copy(
                rows_v.at[b], out_slice(s), osems[b]
            ).wait()

        for p in range(4):
            start_idx(p, p)
        wait_idx(0, 0)
        start_gather(0)
        wait_idx(1, 1)
        start_gather(1)

        def body(g, carry):
            s0 = g * 2

            def do_block(b, ib_cur, ib_next):
                s = s0 + b
                wait_gather(ib_cur)

                @pl.when(g > 0)
                def _():
                    wait_out(s - 2, ib_next)

                start_out(s, ib_cur)

                @pl.when(g < S // 2 - 1)
                def _():
                    wait_idx(s + 2, ib_next)
                    start_gather(ib_next)

                @pl.when(g < S // 2 - 2)
                def _():
                    start_idx(s + 4, ib_cur)

            @pl.when(lax.rem(g, 2) == 0)
            def _():
                do_block(0, 0, 2)
                do_block(1, 1, 3)

            @pl.when(lax.rem(g, 2) == 1)
            def _():
                do_block(0, 2, 0)
                do_block(1, 3, 1)

            return carry

        lax.fori_loop(0, S // 2, body, 0)
        wait_out(S - 2, 0)
        wait_out(S - 1, 1)

    return gather_kernel


def kernel(inputs, embeddings):
    V, D = embeddings.shape
    R, S = inputs.shape
    NBF = V // 128
    Vp = NBF * 128 + 128
    tail = embeddings[NBF * 128:]
    tail_pad = jnp.pad(
        tail, ((0, 128 - tail.shape[0]), (0, 128 - D))
    )
    table_pad = _make_detile(V, D)(embeddings.T, tail_pad)
    idx_t = inputs.T.astype(jnp.int32).reshape(S * R)
    out_sr = _make_gather(V, D, R, S, Vp)(idx_t, table_pad)
    return out_sr[:, :, :D].transpose(1, 0, 2)


# speedup vs baseline: 1.9397x; 1.9397x over previous
import functools

import jax
import jax.numpy as jnp
from jax import lax
from jax.experimental import pallas as pl
from jax.experimental.pallas import tpu as pltpu
from jax.experimental.pallas import tpu_sc as plsc


@functools.lru_cache(maxsize=None)
def _make_detile(V, D):
    info = plsc.get_sparse_core_info()
    NC, NS, L = info.num_cores, info.num_subcores, info.num_lanes
    NW = NC * NS
    NBF = V // 128
    Vp = NBF * 128 + 128
    mesh = plsc.VectorSubcoreMesh(core_axis_name="c", subcore_axis_name="s")

    @functools.partial(
        pl.kernel,
        mesh=mesh,
        out_type=jax.ShapeDtypeStruct((Vp, 128), jnp.float32),
        scratch_types=[
            pltpu.VMEM((2, D, 128), jnp.float32),
            pltpu.VMEM((2, 128, 128), jnp.float32),
            [pltpu.SemaphoreType.DMA] * 2,
            [pltpu.SemaphoreType.DMA] * 2,
        ],
        compiler_params=pltpu.CompilerParams(use_tc_tiling_on_sc=True, needs_layout_passes=False),
    )
    def detile_kernel(emb_t, tail_pad, out, stage_v, tout_v, gsems, osems):
        wid = lax.axis_index("s") * NC + lax.axis_index("c")

        @pl.when(wid == 0)
        def _():
            pltpu.sync_copy(tail_pad, tout_v.at[0])
            pltpu.sync_copy(tout_v.at[0], out.at[pl.ds(NBF * 128, 128)])

        nb = (NBF + NW - 1) // NW

        def col(g):
            return lax.rem(wid + g * NW, NBF)

        def src_slice(b):
            return emb_t.at[:, pl.ds(b * 128, 128)]

        def start_in(b, slot):
            pltpu.async_copy(src_slice(b), stage_v.at[slot], gsems[slot])

        def wait_in(b, slot):
            pltpu.make_async_copy(
                src_slice(b), stage_v.at[slot], gsems[slot]
            ).wait()

        def dst_slice(b):
            return out.at[pl.ds(b * 128, 128)]

        def start_out(b, slot):
            pltpu.async_copy(tout_v.at[slot], dst_slice(b), osems[slot])

        def wait_out(b, slot):
            pltpu.make_async_copy(
                tout_v.at[slot], dst_slice(b), osems[slot]
            ).wait()

        lane = lax.iota(jnp.int32, L)
        rowv = [lane + (fg * L) for fg in range(D // L)]
        zero = jnp.zeros((L,), jnp.int32)

        def transpose(slot):
            @plsc.parallel_loop(0, 128, step=1, unroll=8)
            def _(l):
                colv = zero + l
                for fg in range(D // L):
                    vals = plsc.load_gather(stage_v.at[slot], [rowv[fg], colv])
                    tout_v[slot, l, pl.ds(fg * L, L)] = vals

        start_in(col(0), 0)
        start_in(col(1), 1)

        assert nb % 2 == 0 or nb % 2 == 1
        n_groups = nb // 2

        def body(grp, carry):
            g0 = grp * 2
            for b in range(2):
                g = g0 + b
                wait_in(col(g), b)

                @pl.when(grp > 0)
                def _():
                    wait_out(col(g - 2), b)

                transpose(b)
                start_out(col(g), b)

                @pl.when(g + 2 < nb)
                def _():
                    start_in(col(g + 2), b)

            return carry

        lax.fori_loop(0, n_groups, body, 0)
        if nb % 2 == 1:
            g = nb - 1
            wait_in(col(g), 0)
            wait_out(col(g - 2), 0)
            transpose(0)
            start_out(col(g), 0)
            wait_out(col(g - 1), 1)
            wait_out(col(g), 0)
        else:
            wait_out(col(nb - 2), 0)
            wait_out(col(nb - 1), 1)

    return detile_kernel


@functools.lru_cache(maxsize=None)
def _make_gather(V, D, R, S, Vp):
    info = plsc.get_sparse_core_info()
    NC, NS = info.num_cores, info.num_subcores
    NW = NC * NS
    assert R % (NW * 128) == 0
    mesh = plsc.VectorSubcoreMesh(core_axis_name="c", subcore_axis_name="s")

    @functools.partial(
        pl.kernel,
        mesh=mesh,
        out_type=jax.ShapeDtypeStruct((S, R, 128), jnp.float32),
        scratch_types=[
            pltpu.VMEM((4, 128), jnp.int32),
            pltpu.VMEM((4, 128, 128), jnp.float32),
            [pltpu.SemaphoreType.DMA] * 4,
            [pltpu.SemaphoreType.DMA] * 4,
            [pltpu.SemaphoreType.DMA] * 4,
        ],
        compiler_params=pltpu.CompilerParams(use_tc_tiling_on_sc=True, needs_layout_passes=False),
    )
    def gather_kernel(idx_hbm, table_hbm, out_hbm, idx_v, rows_v, isems,
                      gsems, osems):
        wid = lax.axis_index("s") * NC + lax.axis_index("c")

        def idx_slice(s):
            return idx_hbm.at[pl.ds(s * R + wid * 128, 128)]

        def start_idx(s, ib):
            pltpu.async_copy(idx_slice(s), idx_v.at[ib], isems[ib])

        def wait_idx(s, ib):
            pltpu.make_async_copy(idx_slice(s), idx_v.at[ib], isems[ib]).wait()

        def start_gather(ib):
            pltpu.async_copy(
                table_hbm.at[idx_v.at[ib]], rows_v.at[ib], gsems[ib]
            )

        def wait_gather(ib):
            pltpu.make_async_copy(
                table_hbm.at[idx_v.at[ib]], rows_v.at[ib], gsems[ib]
            ).wait()

        def out_slice(s):
            return out_hbm.at[s, pl.ds(wid * 128, 128), :]

        def start_out(s, b):
            pltpu.async_copy(rows_v.at[b], out_slice(s), osems[b])

        def wait_out(s, b):
            pltpu.make_async_copy(
                rows_v.at[b], out_slice(s), osems[b]
            ).wait()

        for p in range(4):
            start_idx(p, p)
        wait_idx(0, 0)
        start_gather(0)
        wait_idx(1, 1)
        start_gather(1)

        def body(g, carry):
            s0 = g * 2

            def do_block(b, ib_cur, ib_next):
                s = s0 + b
                wait_gather(ib_cur)

                @pl.when(g > 0)
                def _():
                    wait_out(s - 2, ib_next)

                start_out(s, ib_cur)

                @pl.when(g < S // 2 - 1)
                def _():
                    wait_idx(s + 2, ib_next)
                    start_gather(ib_next)

                @pl.when(g < S // 2 - 2)
                def _():
                    start_idx(s + 4, ib_cur)

            @pl.when(lax.rem(g, 2) == 0)
            def _():
                do_block(0, 0, 2)
                do_block(1, 1, 3)

            @pl.when(lax.rem(g, 2) == 1)
            def _():
                do_block(0, 2, 0)
                do_block(1, 3, 1)

            return carry

        lax.fori_loop(0, S // 2, body, 0)
        wait_out(S - 2, 0)
        wait_out(S - 1, 1)

    return gather_kernel


def kernel(inputs, embeddings):
    V, D = embeddings.shape
    R, S = inputs.shape
    NBF = V // 128
    Vp = NBF * 128 + 128
    tail = embeddings[NBF * 128:]
    tail_pad = jnp.pad(
        tail, ((0, 128 - tail.shape[0]), (0, 128 - D))
    )
    table_pad = _make_detile(V, D)(embeddings.T, tail_pad)
    idx_t = inputs.T.astype(jnp.int32).reshape(S * R)
    out_sr = _make_gather(V, D, R, S, Vp)(idx_t, table_pad)
    return out_sr[:, :, :D].transpose(1, 0, 2)
